# baseline (speedup 1.0000x reference)
import functools

import jax
import jax.numpy as jnp
from jax import lax
from jax.experimental import pallas as pl
from jax.experimental.pallas import tpu as pltpu
from jax.experimental.pallas import tpu_sc as plsc

W = 1024
EPS = 1e-10
INVALID = -1


def _sc_gather_rows(d2, t2, rowd, rowt):
    NP = rowd.shape[0]
    PW = NP // 32
    mesh = plsc.VectorSubcoreMesh(core_axis_name="c", subcore_axis_name="s")

    @functools.partial(
        pl.kernel,
        out_type=[jax.ShapeDtypeStruct((NP, 128), jnp.float32),
                  jax.ShapeDtypeStruct((NP, 128), jnp.float32)],
        mesh=mesh,
        scratch_types=[pltpu.VMEM((PW,), jnp.int32),
                       pltpu.VMEM((PW,), jnp.int32),
                       pltpu.VMEM((PW, 128), jnp.float32),
                       pltpu.VMEM((PW, 128), jnp.float32),
                       pltpu.SemaphoreType.DMA,
                       pltpu.SemaphoreType.DMA],
    )
    def k(d_hbm, t_hbm, rowd_hbm, rowt_hbm, qr_hbm, pr_hbm,
          idxd, idxt, bufd, buft, sem1, sem2):
        wid = lax.axis_index("s") * 2 + lax.axis_index("c")
        base = wid * PW
        pltpu.sync_copy(rowd_hbm.at[pl.ds(base, PW)], idxd)
        pltpu.sync_copy(rowt_hbm.at[pl.ds(base, PW)], idxt)
        cp1 = pltpu.async_copy(d_hbm.at[idxd], bufd, sem1)
        cp2 = pltpu.async_copy(t_hbm.at[idxt], buft, sem2)
        cp1.wait()
        cp2.wait()
        pltpu.sync_copy(bufd, qr_hbm.at[pl.ds(base, PW)])
        pltpu.sync_copy(buft, pr_hbm.at[pl.ds(base, PW)])

    return k(d2, t2, rowd, rowt)


def _k1_body(*refs, V, C, S, CP):
    sr_ref, st_ref = refs[2 * S:]
    Kp1 = refs[S].shape[1]
    K = Kp1 - 1

    for s in range(S):
        d = refs[s][0]
        t = refs[S + s][0]
        t8 = t[:K, :]
        r = jnp.maximum(t8 - d, 0.0)
        for cl in range(CP):
            c = s * CP + cl
            rs = r[:, cl * W:(cl + 1) * W]
            ts = t[:, cl * W:(cl + 1) * W]
            if (c + 1) * W > V:
                m = (lax.broadcasted_iota(jnp.int32, (Kp1, W), 1)
                     + c * W) < V
                ts = jnp.where(m, ts, 0.0)
                rs = jnp.where(m[:K], rs, 0.0)
            sr_ref[0, :, c:c + 1] = rs.sum(-1, keepdims=True)
            st_ref[0, :, c:c + 1] = ts.sum(-1, keepdims=True)


def _run_k1(draft, target):
    B, K, V = draft.shape
    C = pl.cdiv(V, W)
    S = 7
    CP = C // S
    PV = CP * W
    d_specs = [pl.BlockSpec((1, K, PV), functools.partial(
        lambda b, ss: (b, 0, ss), ss=s)) for s in range(S)]
    t_specs = [pl.BlockSpec((1, K + 1, PV), functools.partial(
        lambda b, ss: (b, 0, ss), ss=s)) for s in range(S)]
    return pl.pallas_call(
        functools.partial(_k1_body, V=V, C=C, S=S, CP=CP),
        grid=(B,),
        in_specs=d_specs + t_specs,
        out_specs=[
            pl.BlockSpec((1, K, C), lambda b: (b, 0, 0)),
            pl.BlockSpec((1, K + 1, C), lambda b: (b, 0, 0)),
        ],
        out_shape=[
            jax.ShapeDtypeStruct((B, K, C), jnp.float32),
            jax.ShapeDtypeStruct((B, K + 1, C), jnp.float32),
        ],
    )(*([draft] * S), *([target] * S))


def _k2_body(sr_ref, st_ref, u_ref, qr_ref, pr_ref, ohd_ref, oht_ref, ua_ref,
             cstar_ref, base_ref, thr_ref, flag_ref, na_ref):
    N, C = sr_ref.shape
    sr = sr_ref[...]
    st = st_ref[...]
    u = u_ref[...]

    s_tot = sr.sum(-1, keepdims=True)
    kpos = lax.broadcasted_iota(jnp.int32, (N, 1), 0) % 9
    use_r = (s_tot > EPS) & (kpos < 8)
    sel = jnp.where(use_r, sr, st)
    thr = jnp.where(use_r, u * s_tot, u)

    ci_ = lax.broadcasted_iota(jnp.int32, (C, C), 0)
    cj_ = lax.broadcasted_iota(jnp.int32, (C, C), 1)
    utri = (ci_ <= cj_).astype(jnp.float32)
    cc = jax.lax.dot_general(sel, utri, (((1,), (0,)), ((), ())),
                             preferred_element_type=jnp.float32)
    cstar = (cc < thr).astype(jnp.int32).sum(-1, keepdims=True)
    cstar = jnp.minimum(cstar, C - 1)
    cj = lax.broadcasted_iota(jnp.int32, (N, C), 1)
    base = jnp.where(cj < cstar, sel, 0.0).sum(-1, keepdims=True)

    cstar_ref[...] = cstar
    base_ref[...] = base
    thr_ref[...] = thr
    flag_ref[...] = use_r.astype(jnp.int32)

    q = (qr_ref[...] * ohd_ref[...]).sum(-1)
    p = (pr_ref[...] * oht_ref[...]).sum(-1)
    ua = ua_ref[...]
    acc_prob = jnp.minimum(1.0, p / jnp.maximum(q, EPS))
    rejected = (ua > acc_prob).astype(jnp.float32)
    K = rejected.shape[1]
    ki_ = lax.broadcasted_iota(jnp.int32, (K, K), 0)
    kj_ = lax.broadcasted_iota(jnp.int32, (K, K), 1)
    ktri = (ki_ <= kj_).astype(jnp.float32)
    cumrej = jax.lax.dot_general(rejected, ktri, (((1,), (0,)), ((), ())),
                                 preferred_element_type=jnp.float32)
    na = (cumrej == 0.0).astype(jnp.int32).sum(-1, keepdims=True)
    na_ref[...] = na


def _run_k2(sr_pairs, st_pairs, u_col, qr, pr, ohd, oht, ua):
    B = ua.shape[0]
    N = sr_pairs.shape[0]
    return pl.pallas_call(
        _k2_body,
        out_shape=[
            jax.ShapeDtypeStruct((N, 1), jnp.int32),
            jax.ShapeDtypeStruct((N, 1), jnp.float32),
            jax.ShapeDtypeStruct((N, 1), jnp.float32),
            jax.ShapeDtypeStruct((N, 1), jnp.int32),
            jax.ShapeDtypeStruct((B, 1), jnp.int32),
        ],
    )(sr_pairs, st_pairs, u_col, qr, pr, ohd, oht, ua)


def _k3_body(cs_ref, d_ref, t_ref, thr_ref, base_ref, flag_ref, na_ref,
             ids_ref, out_ref, *, V):
    b = pl.program_id(0)

    thr = thr_ref[0]
    base = base_ref[0]
    flag = flag_ref[0]

    rows = []
    cbase = []
    for k in range(9):
        cstar_k = cs_ref[b * 9 + k]
        t = t_ref[k][0:1, k:k + 1, :][0]
        if k < 8:
            d = d_ref[k][0:1, k:k + 1, :][0]
            fk = flag[0:1, k:k + 1]
            vals = jnp.where(fk > 0, jnp.maximum(t - d, 0.0), t)
        else:
            vals = t
        li = lax.broadcasted_iota(jnp.int32, (1, W), 1) + cstar_k * W
        vals = jnp.where(li < V, vals, 0.0)
        rows.append(vals)
        cbase.append(cstar_k * W)
    vals9 = jnp.concatenate(rows, axis=0)

    wi_ = lax.broadcasted_iota(jnp.int32, (W, W), 0)
    wj_ = lax.broadcasted_iota(jnp.int32, (W, W), 1)
    utri = (wi_ <= wj_).astype(jnp.float32)
    cum9 = jax.lax.dot_general(vals9, utri, (((1,), (0,)), ((), ())),
                               preferred_element_type=jnp.float32)
    toks = []
    for k in range(9):
        cum_k = cum9[k:k + 1, :] + base[0:1, k:k + 1]
        cnt_k = (cum_k < thr[0:1, k:k + 1]).astype(jnp.int32).sum(
            -1, keepdims=True)
        toks.append(jnp.minimum(cbase[k] + cnt_k, V - 1))
    rec = jnp.concatenate(toks, axis=-1)

    ids_ext = jnp.concatenate(
        [ids_ref[0], jnp.zeros((1, 1), jnp.int32)], axis=-1)
    pos = lax.broadcasted_iota(jnp.int32, (1, 9), 1)
    na = na_ref[0]
    out_ref[0] = jnp.where(pos < na, ids_ext,
                           jnp.where(pos == na, rec,
                                     jnp.full((1, 9), INVALID, jnp.int32)))


def _run_k3(cstar_flat, draft, target, thr, base, flag, na, ids):
    B, K, V = draft.shape
    d_specs = [
        pl.BlockSpec((1, K, W), functools.partial(
            lambda b, cs, kk: (b, 0, cs[b * 9 + kk]), kk=k))
        for k in range(8)
    ]
    t_specs = [
        pl.BlockSpec((1, K + 1, W), functools.partial(
            lambda b, cs, kk: (b, 0, cs[b * 9 + kk]), kk=k))
        for k in range(9)
    ]
    grid_spec = pltpu.PrefetchScalarGridSpec(
        num_scalar_prefetch=1,
        grid=(B,),
        in_specs=d_specs + t_specs + [
            pl.BlockSpec((1, 1, 9), lambda b, cs: (b, 0, 0)),
            pl.BlockSpec((1, 1, 9), lambda b, cs: (b, 0, 0)),
            pl.BlockSpec((1, 1, 9), lambda b, cs: (b, 0, 0)),
            pl.BlockSpec((1, 1, 1), lambda b, cs: (b, 0, 0)),
            pl.BlockSpec((1, 1, 8), lambda b, cs: (b, 0, 0)),
        ],
        out_specs=pl.BlockSpec((1, 1, 9), lambda b, cs: (b, 0, 0)),
    )

    def body(cs_ref, *refs):
        return _k3_body(cs_ref, refs[0:8], refs[8:17], *refs[17:], V=V)

    out = pl.pallas_call(
        body,
        grid_spec=grid_spec,
        out_shape=jax.ShapeDtypeStruct((B, 1, 9), jnp.int32),
    )(cstar_flat, *([draft] * 8), *([target] * 9),
      thr, base, flag, na, ids)
    return out.reshape(B, 9)


def kernel(draft_probs, target_probs, uniform_accept, uniform_sample,
           draft_token_ids):
    B, K, V = draft_probs.shape
    C = pl.cdiv(V, W)

    prn = jnp.arange(B * K, dtype=jnp.int32)
    idsf = draft_token_ids.reshape(-1)
    flatd = prn * V + idsf
    flatt = (prn + prn // K) * V + idsf
    qrows, prows = _sc_gather_rows(
        draft_probs.reshape(B * K * V // 128, 128),
        target_probs.reshape(B * (K + 1) * V // 128, 128),
        flatd // 128, flatt // 128)
    lane = jnp.arange(128, dtype=jnp.int32)[None, :]
    ohd = (flatd[:, None] % 128 == lane).astype(jnp.float32)
    oht = (flatt[:, None] % 128 == lane).astype(jnp.float32)

    srT, stT = _run_k1(draft_probs, target_probs)

    sr_pairs = jnp.concatenate(
        [srT, jnp.zeros((B, 1, C), jnp.float32)], axis=1).reshape(B * (K + 1), C)
    st_pairs = stT.reshape(B * (K + 1), C)
    u_col = uniform_sample.reshape(B * (K + 1), 1)

    cstar, base, thr, flag, na = _run_k2(sr_pairs, st_pairs, u_col,
                                         qrows.reshape(B, K, 128),
                                         prows.reshape(B, K, 128),
                                         ohd.reshape(B, K, 128),
                                         oht.reshape(B, K, 128),
                                         uniform_accept)

    thr9 = thr.reshape(B, 1, K + 1)
    base9 = base.reshape(B, 1, K + 1)
    flag9 = flag.reshape(B, 1, K + 1)
    na9 = na.reshape(B, 1, 1)
    ids9 = draft_token_ids.reshape(B, 1, K)
    cstar_flat = cstar.reshape(B * (K + 1))

    return _run_k3(cstar_flat, draft_probs, target_probs,
                   thr9, base9, flag9, na9, ids9)


# speedup vs baseline: 10.5568x; 10.5568x over previous
import functools

import jax
import jax.numpy as jnp
from jax import lax
from jax.experimental import pallas as pl
from jax.experimental.pallas import tpu as pltpu

W = 1024
EPS = 1e-10
INVALID = -1


def _k1_body(ids_ref, *refs, V, C, S, CP):
    sr_ref, st_ref, q_ref, p_ref = refs[2 * S:]
    Kp1 = refs[S].shape[1]
    K = Kp1 - 1
    PV = CP * W
    ids = ids_ref[0]

    sr_ref[0, K:K + 1, :] = jnp.zeros((1, C), jnp.float32)
    qacc = jnp.zeros((K, 1), jnp.float32)
    pacc = jnp.zeros((K, 1), jnp.float32)
    for s in range(S):
        d = refs[s][0]
        t = refs[S + s][0]
        t8 = t[:K, :]
        r = jnp.maximum(t8 - d, 0.0)
        for cl in range(CP):
            c = s * CP + cl
            rs = r[:, cl * W:(cl + 1) * W]
            ts = t[:, cl * W:(cl + 1) * W]
            if (c + 1) * W > V:
                m = (lax.broadcasted_iota(jnp.int32, (Kp1, W), 1)
                     + c * W) < V
                ts = jnp.where(m, ts, 0.0)
                rs = jnp.where(m[:K], rs, 0.0)
            sr_ref[0, :K, c:c + 1] = rs.sum(-1, keepdims=True)
            st_ref[0, :, c:c + 1] = ts.sum(-1, keepdims=True)
        li = lax.broadcasted_iota(jnp.int32, (K, PV), 1) + s * PV
        m = li == ids
        qacc = qacc + jnp.where(m, d, 0.0).sum(-1, keepdims=True)
        pacc = pacc + jnp.where(m, t8, 0.0).sum(-1, keepdims=True)
    q_ref[0] = qacc
    p_ref[0] = pacc


def _run_k1(draft, target, ids):
    B, K, V = draft.shape
    C = pl.cdiv(V, W)
    S = 7
    CP = C // S
    PV = CP * W
    d_specs = [pl.BlockSpec((1, K, PV), functools.partial(
        lambda b, ss: (b, 0, ss), ss=s)) for s in range(S)]
    t_specs = [pl.BlockSpec((1, K + 1, PV), functools.partial(
        lambda b, ss: (b, 0, ss), ss=s)) for s in range(S)]
    return pl.pallas_call(
        functools.partial(_k1_body, V=V, C=C, S=S, CP=CP),
        grid=(B,),
        in_specs=[pl.BlockSpec((1, K, 1), lambda b: (b, 0, 0))]
        + d_specs + t_specs,
        out_specs=[
            pl.BlockSpec((1, K + 1, C), lambda b: (b, 0, 0)),
            pl.BlockSpec((1, K + 1, C), lambda b: (b, 0, 0)),
            pl.BlockSpec((1, K, 1), lambda b: (b, 0, 0)),
            pl.BlockSpec((1, K, 1), lambda b: (b, 0, 0)),
        ],
        out_shape=[
            jax.ShapeDtypeStruct((B, K + 1, C), jnp.float32),
            jax.ShapeDtypeStruct((B, K + 1, C), jnp.float32),
            jax.ShapeDtypeStruct((B, K, 1), jnp.float32),
            jax.ShapeDtypeStruct((B, K, 1), jnp.float32),
        ],
    )(ids.reshape(B, K, 1), *([draft] * S), *([target] * S))


def _k2_body(sr_ref, st_ref, u_ref, q_ref, p_ref, ua_ref,
             cstar_ref, base_ref, thr_ref, flag_ref, na_ref):
    N, C = sr_ref.shape
    sr = sr_ref[...]
    st = st_ref[...]
    u = u_ref[...]

    s_tot = sr.sum(-1, keepdims=True)
    kpos = lax.broadcasted_iota(jnp.int32, (N, 1), 0) % 9
    use_r = (s_tot > EPS) & (kpos < 8)
    sel = jnp.where(use_r, sr, st)
    thr = jnp.where(use_r, u * s_tot, u)

    ci_ = lax.broadcasted_iota(jnp.int32, (C, C), 0)
    cj_ = lax.broadcasted_iota(jnp.int32, (C, C), 1)
    utri = (ci_ <= cj_).astype(jnp.float32)
    cc = jax.lax.dot_general(sel, utri, (((1,), (0,)), ((), ())),
                             preferred_element_type=jnp.float32)
    cstar = (cc < thr).astype(jnp.int32).sum(-1, keepdims=True)
    cstar = jnp.minimum(cstar, C - 1)
    cj = lax.broadcasted_iota(jnp.int32, (N, C), 1)
    base = jnp.where(cj < cstar, sel, 0.0).sum(-1, keepdims=True)

    cstar_ref[...] = cstar
    base_ref[...] = base
    thr_ref[...] = thr
    flag_ref[...] = use_r.astype(jnp.int32)

    q = q_ref[...]
    p = p_ref[...]
    ua = ua_ref[...]
    acc_prob = jnp.minimum(1.0, p / jnp.maximum(q, EPS))
    rejected = (ua > acc_prob).astype(jnp.float32)
    K = rejected.shape[1]
    ki_ = lax.broadcasted_iota(jnp.int32, (K, K), 0)
    kj_ = lax.broadcasted_iota(jnp.int32, (K, K), 1)
    ktri = (ki_ <= kj_).astype(jnp.float32)
    cumrej = jax.lax.dot_general(rejected, ktri, (((1,), (0,)), ((), ())),
                                 preferred_element_type=jnp.float32)
    na = (cumrej == 0.0).astype(jnp.int32).sum(-1, keepdims=True)
    na_ref[...] = na


def _run_k2(sr_pairs, st_pairs, u_col, q, p, ua):
    B = q.shape[0]
    N = sr_pairs.shape[0]
    return pl.pallas_call(
        _k2_body,
        out_shape=[
            jax.ShapeDtypeStruct((N, 1), jnp.int32),
            jax.ShapeDtypeStruct((N, 1), jnp.float32),
            jax.ShapeDtypeStruct((N, 1), jnp.float32),
            jax.ShapeDtypeStruct((N, 1), jnp.int32),
            jax.ShapeDtypeStruct((B, 1), jnp.int32),
        ],
    )(sr_pairs, st_pairs, u_col, q, p, ua)


def _k3_body(cs_ref, d_ref, t_ref, thr_ref, base_ref, flag_ref, na_ref,
             ids_ref, out_ref, *, V, G):
    bs = pl.program_id(0)

    rows = []
    cbase = []
    for j in range(G):
        flag = flag_ref[j]
        for k in range(9):
            cstar_k = cs_ref[(bs * G + j) * 9 + k]
            t = t_ref[j * 9 + k][0:1, k:k + 1, :][0]
            if k < 8:
                d = d_ref[j * 8 + k][0:1, k:k + 1, :][0]
                fk = flag[0:1, k:k + 1]
                vals = jnp.where(fk > 0, jnp.maximum(t - d, 0.0), t)
            else:
                vals = t
            li = lax.broadcasted_iota(jnp.int32, (1, W), 1) + cstar_k * W
            vals = jnp.where(li < V, vals, 0.0)
            rows.append(vals)
            cbase.append(cstar_k * W)
    valsg = jnp.concatenate(rows, axis=0)

    wi_ = lax.broadcasted_iota(jnp.int32, (W, W), 0)
    wj_ = lax.broadcasted_iota(jnp.int32, (W, W), 1)
    utri = (wi_ <= wj_).astype(jnp.float32)
    cumg = jax.lax.dot_general(valsg, utri, (((1,), (0,)), ((), ())),
                               preferred_element_type=jnp.float32)
    for j in range(G):
        thr = thr_ref[j]
        base = base_ref[j]
        toks = []
        for k in range(9):
            rr = j * 9 + k
            cum_k = cumg[rr:rr + 1, :] + base[0:1, k:k + 1]
            cnt_k = (cum_k < thr[0:1, k:k + 1]).astype(jnp.int32).sum(
                -1, keepdims=True)
            toks.append(jnp.minimum(cbase[rr] + cnt_k, V - 1))
        rec = jnp.concatenate(toks, axis=-1)

        ids_ext = jnp.concatenate(
            [ids_ref[j], jnp.zeros((1, 1), jnp.int32)], axis=-1)
        pos = lax.broadcasted_iota(jnp.int32, (1, 9), 1)
        na = na_ref[j]
        out_ref[j] = jnp.where(pos < na, ids_ext,
                               jnp.where(pos == na, rec,
                                         jnp.full((1, 9), INVALID,
                                                  jnp.int32)))


def _run_k3(cstar_flat, draft, target, thr, base, flag, na, ids):
    B, K, V = draft.shape
    G = 2
    d_specs = [
        pl.BlockSpec((1, K, W), functools.partial(
            lambda b, cs, jj, kk: (b * G + jj, 0, cs[(b * G + jj) * 9 + kk]),
            jj=j, kk=k))
        for j in range(G) for k in range(8)
    ]
    t_specs = [
        pl.BlockSpec((1, K + 1, W), functools.partial(
            lambda b, cs, jj, kk: (b * G + jj, 0, cs[(b * G + jj) * 9 + kk]),
            jj=j, kk=k))
        for j in range(G) for k in range(9)
    ]
    grid_spec = pltpu.PrefetchScalarGridSpec(
        num_scalar_prefetch=1,
        grid=(B // G,),
        in_specs=d_specs + t_specs + [
            pl.BlockSpec((G, 1, 9), lambda b, cs: (b, 0, 0)),
            pl.BlockSpec((G, 1, 9), lambda b, cs: (b, 0, 0)),
            pl.BlockSpec((G, 1, 9), lambda b, cs: (b, 0, 0)),
            pl.BlockSpec((G, 1, 1), lambda b, cs: (b, 0, 0)),
            pl.BlockSpec((G, 1, 8), lambda b, cs: (b, 0, 0)),
        ],
        out_specs=pl.BlockSpec((G, 1, 9), lambda b, cs: (b, 0, 0)),
    )

    def body(cs_ref, *refs):
        return _k3_body(cs_ref, refs[0:8 * G], refs[8 * G:17 * G],
                        *refs[17 * G:], V=V, G=G)

    out = pl.pallas_call(
        body,
        grid_spec=grid_spec,
        out_shape=jax.ShapeDtypeStruct((B, 1, 9), jnp.int32),
    )(cstar_flat, *([draft] * (8 * G)), *([target] * (9 * G)),
      thr, base, flag, na, ids)
    return out.reshape(B, 9)


def kernel(draft_probs, target_probs, uniform_accept, uniform_sample,
           draft_token_ids):
    B, K, V = draft_probs.shape
    C = pl.cdiv(V, W)
    srT, stT, q3, p3 = _run_k1(draft_probs, target_probs, draft_token_ids)
    q = q3.reshape(B, K)
    p = p3.reshape(B, K)

    sr_pairs = srT.reshape(B * (K + 1), C)
    st_pairs = stT.reshape(B * (K + 1), C)
    u_col = uniform_sample.reshape(B * (K + 1), 1)

    cstar, base, thr, flag, na = _run_k2(sr_pairs, st_pairs, u_col, q, p,
                                         uniform_accept)

    thr9 = thr.reshape(B, 1, K + 1)
    base9 = base.reshape(B, 1, K + 1)
    flag9 = flag.reshape(B, 1, K + 1)
    na9 = na.reshape(B, 1, 1)
    ids9 = draft_token_ids.reshape(B, 1, K)
    cstar_flat = cstar.reshape(B * (K + 1))

    return _run_k3(cstar_flat, draft_probs, target_probs,
                   thr9, base9, flag9, na9, ids9)
